# baseline (speedup 1.0000x reference)
import functools

import jax
import jax.numpy as jnp
from jax import lax
from jax.experimental import pallas as pl
from jax.experimental.pallas import tpu as pltpu
from jax.experimental.pallas import tpu_sc as plsc

_N, _E, _M = 2048, 8192, 64
_H, _HE = 256, 128
_D = 512
_PW = 384
_ACC_ROWS = 2176
_TRASH = _N
_NC, _NS = 2, 16

_SC_MESH = dict(core_axis_name="c", subcore_axis_name="s",
                num_cores=_NC, num_subcores=_NS)


def _lrelu(x):
    return jnp.where(x >= 0, x, 0.01 * x)


def _gru_math(x, h, wih, whh, bih, bhh):
    gi = jnp.dot(x, wih, preferred_element_type=jnp.float32) + bih
    gh = jnp.dot(h, whh, preferred_element_type=jnp.float32) + bhh
    r = jax.nn.sigmoid(gi[:, :_H] + gh[:, :_H])
    z = jax.nn.sigmoid(gi[:, _H:2 * _H] + gh[:, _H:2 * _H])
    n = jnp.tanh(gi[:, 2 * _H:] + r * gh[:, 2 * _H:])
    return (1.0 - z) * n + z * h


def _proj_body(x_ref, w_ref, b_ref, o_ref):
    o_ref[...] = _lrelu(
        jnp.dot(x_ref[...], w_ref[...], preferred_element_type=jnp.float32)
        + b_ref[...])


def _proj(x, w, b, bm):
    n, k = x.shape
    c = w.shape[1]
    return pl.pallas_call(
        _proj_body,
        grid=(n // bm,),
        in_specs=[pl.BlockSpec((bm, k), lambda i: (i, 0)),
                  pl.BlockSpec((k, c), lambda i: (0, 0)),
                  pl.BlockSpec((1, c), lambda i: (0, 0))],
        out_specs=pl.BlockSpec((bm, c), lambda i: (i, 0)),
        out_shape=jax.ShapeDtypeStruct((n, c), jnp.float32),
    )(x, w, b.reshape(1, c))


def _node_table_body(h_ref, w_ref, o_ref):
    o_ref[...] = jnp.dot(h_ref[...], w_ref[0],
                         preferred_element_type=jnp.float32)


def _node_table(h, w2, bm=256):
    return pl.pallas_call(
        _node_table_body,
        grid=(2, _N // bm),
        in_specs=[pl.BlockSpec((bm, _H), lambda g, i: (i, 0)),
                  pl.BlockSpec((1, _H, _D), lambda g, i: (g, 0, 0))],
        out_specs=pl.BlockSpec((bm, _D), lambda g, i: (g * (_N // bm) + i, 0)),
        out_shape=jax.ShapeDtypeStruct((2 * _N, _D), jnp.float32),
    )(h, w2)


def _edge_fuse_body(g0_ref, g1_ref, e_ref, we_ref, b_ref,
                    pay_ref, eo_ref, msum_ref):
    z = (g0_ref[...] + g1_ref[...]
         + jnp.dot(e_ref[...], we_ref[...], preferred_element_type=jnp.float32)
         + b_ref[...])
    msg = _lrelu(z[:, :_H])
    eo_ref[...] = _lrelu(z[:, _H:_H + _HE])
    w = jnp.exp(z[:, _H + _HE:_H + _HE + 1])
    bm = msg.shape[0]
    pay_ref[...] = jnp.concatenate(
        [w * msg, w, jnp.zeros((bm, _PW - _H - 1), jnp.float32)], axis=1)

    @pl.when(pl.program_id(0) == 0)
    def _():
        msum_ref[...] = jnp.zeros_like(msum_ref)
    msum_ref[...] += jnp.sum(msg, axis=0, keepdims=True)


def _edge_fuse(gat, e, we, b400, bm=512):
    nb = _E // bm
    return pl.pallas_call(
        _edge_fuse_body,
        grid=(nb,),
        in_specs=[pl.BlockSpec((bm, _D), lambda i: (i, 0)),
                  pl.BlockSpec((bm, _D), lambda i: (i + nb, 0)),
                  pl.BlockSpec((bm, _HE), lambda i: (i, 0)),
                  pl.BlockSpec((_HE, _D), lambda i: (0, 0)),
                  pl.BlockSpec((1, _D), lambda i: (0, 0))],
        out_specs=[pl.BlockSpec((bm, _PW), lambda i: (i, 0)),
                   pl.BlockSpec((bm, _HE), lambda i: (i, 0)),
                   pl.BlockSpec((1, _H), lambda i: (0, 0))],
        out_shape=[jax.ShapeDtypeStruct((_E, _PW), jnp.float32),
                   jax.ShapeDtypeStruct((_E, _HE), jnp.float32),
                   jax.ShapeDtypeStruct((1, _H), jnp.float32)],
    )(gat, gat, e, we, b400.reshape(1, _D))


def _nem_matmul_body(nem_ref, p_ref, o_ref):
    @pl.when(pl.program_id(1) == 0)
    def _():
        o_ref[...] = jnp.zeros_like(o_ref)
    o_ref[...] += jnp.dot(nem_ref[...], p_ref[...],
                          preferred_element_type=jnp.float32)


def _tc_scatter(nem, payload, bm=256, bk=512):
    return pl.pallas_call(
        _nem_matmul_body,
        grid=(_N // bm, _E // bk),
        in_specs=[pl.BlockSpec((bm, bk), lambda i, k: (i, k)),
                  pl.BlockSpec((bk, _PW), lambda i, k: (k, 0))],
        out_specs=pl.BlockSpec((bm, _PW), lambda i, k: (i, 0)),
        out_shape=jax.ShapeDtypeStruct((_N, _PW), jnp.float32),
    )(nem, payload)


def _update_body(relu_flag, acc_ref, h_ref, msum_ref,
                 wih_ref, whh_ref, bih_ref, bhh_ref, o_ref):
    acc = acc_ref[...]
    numer = acc[:, :_H]
    den = acc[:, _H:_H + 1]
    ctx = jnp.where(den > 0.0, numer / den, msum_ref[...] * (1.0 / _E))
    hn = _gru_math(ctx, h_ref[...], wih_ref[...], whh_ref[...],
                   bih_ref[...], bhh_ref[...])
    if relu_flag:
        hn = jnp.maximum(hn, 0.0)
    o_ref[...] = hn


def _update(acc, h, msum, wih, whh, bih, bhh, relu_flag, bm=256):
    return pl.pallas_call(
        functools.partial(_update_body, relu_flag),
        grid=(_N // bm,),
        in_specs=[pl.BlockSpec((bm, _PW), lambda i: (i, 0)),
                  pl.BlockSpec((bm, _H), lambda i: (i, 0)),
                  pl.BlockSpec((1, _H), lambda i: (0, 0)),
                  pl.BlockSpec((_H, 3 * _H), lambda i: (0, 0)),
                  pl.BlockSpec((_H, 3 * _H), lambda i: (0, 0)),
                  pl.BlockSpec((1, 3 * _H), lambda i: (0, 0)),
                  pl.BlockSpec((1, 3 * _H), lambda i: (0, 0))],
        out_specs=pl.BlockSpec((bm, _H), lambda i: (i, 0)),
        out_shape=jax.ShapeDtypeStruct((_N, _H), jnp.float32),
    )(acc, h, msum, wih, whh, bih.reshape(1, -1), bhh.reshape(1, -1))


def _readout_body(h_ref, mnm_ref, mask_ref, wq_ref, bq_ref, wk_ref, bk_ref,
                  wv_ref, bv_ref, wih_ref, whh_ref, bih_ref, bhh_ref,
                  wo_ref, bo_ref, ro_ref, a_ref):
    h = h_ref[...]
    mnm = mnm_ref[...]
    mask = mask_ref[...]
    deg = jnp.clip(jnp.sum(mnm, axis=1, keepdims=True), 1.0, None)
    m = jnp.dot(mnm, h, preferred_element_type=jnp.float32) / deg
    kx = jnp.dot(h, wk_ref[...], preferred_element_type=jnp.float32) + bk_ref[...]
    vx = jnp.dot(h, wv_ref[...], preferred_element_type=jnp.float32) + bv_ref[...]
    a = None
    for _ in range(2):
        q = jnp.dot(m, wq_ref[...], preferred_element_type=jnp.float32) + bq_ref[...]
        logits = lax.dot_general(q, kx, (((1,), (1,)), ((), ())),
                                 preferred_element_type=jnp.float32) * (1.0 / 16.0)
        logits = logits + mask
        mx = jnp.max(logits, axis=1, keepdims=True)
        ex = jnp.exp(logits - mx)
        a = ex / jnp.sum(ex, axis=1, keepdims=True)
        c = jnp.dot(a, vx, preferred_element_type=jnp.float32)
        m = _gru_math(c, m, wih_ref[...], whh_ref[...], bih_ref[...],
                      bhh_ref[...])
    ro_ref[...] = jnp.maximum(
        jnp.dot(m, wo_ref[...], preferred_element_type=jnp.float32)
        + bo_ref[...], 0.0)
    a_ref[...] = a


def _readout(h, mnm, mask, p):
    args = [h, mnm, mask,
            p['R_q'][0], p['R_q'][1].reshape(1, -1),
            p['R_k'][0], p['R_k'][1].reshape(1, -1),
            p['R_v'][0], p['R_v'][1].reshape(1, -1),
            p['R_Wih'], p['R_Whh'],
            p['R_bih'].reshape(1, -1), p['R_bhh'].reshape(1, -1),
            p['R_out'][0], p['R_out'][1].reshape(1, -1)]
    return pl.pallas_call(
        _readout_body,
        out_shape=[jax.ShapeDtypeStruct((_M, _H), jnp.float32),
                   jax.ShapeDtypeStruct((_M, _N), jnp.float32)],
    )(*args)


def _gather_body(tbl_hbm, idx_hbm, out_hbm, ibuf, rbuf, sem):
    c = lax.axis_index("c")
    s = lax.axis_index("s")
    wid = s * _NC + c
    per = (2 * _E) // (_NC * _NS)
    base0 = wid * per
    for ch in range(per // 128):
        base = base0 + ch * 128
        pltpu.sync_copy(idx_hbm.at[pl.ds(base, 128)], ibuf)
        pltpu.async_copy(tbl_hbm.at[ibuf], rbuf, sem).wait()
        pltpu.sync_copy(rbuf, out_hbm.at[pl.ds(base, 128)])


def _sc_gather(tbl, idx2):
    return pl.kernel(
        _gather_body,
        out_type=jax.ShapeDtypeStruct((2 * _E, _D), jnp.float32),
        mesh=plsc.VectorSubcoreMesh(**_SC_MESH),
        scratch_types=[pltpu.VMEM((128,), jnp.int32),
                       pltpu.VMEM((128, _D), jnp.float32),
                       pltpu.SemaphoreType.DMA],
    )(tbl, idx2)


def _scatter_body(pay_hbm, iu_hbm, iv_hbm, out_hbm, pbuf, ibuf, acc_sh):
    c = lax.axis_index("c")
    s = lax.axis_index("s")
    zr = _ACC_ROWS // _NS
    zero16 = jnp.zeros((16,), jnp.float32)

    def zrow(r, carry):
        for k in range(_PW // 16):
            pbuf[r, pl.ds(k * 16, 16)] = zero16
        return carry

    lax.fori_loop(0, 128, zrow, 0)
    pltpu.sync_copy(pbuf.at[pl.ds(0, 128)], acc_sh.at[pl.ds(s * zr, 128)])
    pltpu.sync_copy(pbuf.at[pl.ds(0, zr - 128)],
                    acc_sh.at[pl.ds(s * zr + 128, zr - 128)])
    plsc.subcore_barrier()

    per = _E // (_NC * _NS)
    base = c * (_E // _NC) + s * per
    for ch in range(per // 128):
        pltpu.sync_copy(pay_hbm.at[pl.ds(base + ch * 128, 128)], pbuf)
        for ih in (iu_hbm, iv_hbm):
            pltpu.sync_copy(ih.at[pl.ds(base + ch * 128, 128)], ibuf)
            pltpu.sync_copy(pbuf, acc_sh.at[ibuf], add=True)
    plsc.subcore_barrier()
    pltpu.sync_copy(acc_sh.at[pl.ds(s * zr, zr)],
                    out_hbm.at[c].at[pl.ds(s * zr, zr)])


def _sc_scatter(payload, idx_u, idx_v):
    return pl.kernel(
        _scatter_body,
        out_type=jax.ShapeDtypeStruct((_NC, _ACC_ROWS, _PW), jnp.float32),
        mesh=plsc.VectorSubcoreMesh(**_SC_MESH),
        scratch_types=[pltpu.VMEM((128, _PW), jnp.float32),
                       pltpu.VMEM((128,), jnp.int32),
                       pltpu.VMEM_SHARED((_ACC_ROWS, _PW), jnp.float32)],
    )(payload, idx_u, idx_v)


def _pack_layer(p, l):
    wm, bm_ = p['M%d_msg' % l]
    wa, ba = p['M%d_att' % l]
    we, be = p['M%d_edge' % l]

    def cols(lo, hi):
        return jnp.concatenate(
            [wm[lo:hi], we[lo:hi], wa[lo:hi],
             jnp.zeros((hi - lo, _D - _H - _HE - 1), jnp.float32)], axis=1)

    w2 = jnp.stack([cols(0, _H), cols(_H, 2 * _H)])
    wee = cols(2 * _H, 2 * _H + _HE)
    b400 = jnp.concatenate(
        [bm_, be, ba, jnp.zeros((_D - _H - _HE - 1,), jnp.float32)])
    return w2, wee, b400


def kernel(node_features, edge_features, us, vs, mol_node_matrix,
           mol_node_mask, node_edge_matrix, node_edge_mask, params):
    p = params
    us32 = us.astype(jnp.int32)
    vs32 = vs.astype(jnp.int32)
    idx2 = jnp.concatenate([us32, vs32 + _N])
    idx_v = jnp.where(us32 == vs32, _TRASH, vs32)

    h = _proj(node_features, p['FC_N'][0], p['FC_N'][1], bm=256)
    e = _proj(edge_features, p['FC_E'][0], p['FC_E'][1], bm=512)

    for l in range(2):
        w2, wee, b400 = _pack_layer(p, l)
        tbl = _node_table(h, w2)
        gat = _sc_gather(tbl, idx2)
        payload, e, msum = _edge_fuse(gat, e, wee, b400)
        acc = _tc_scatter(node_edge_matrix, payload)
        h = _update(acc, h, msum,
                    p['U%d_Wih' % l], p['U%d_Whh' % l],
                    p['U%d_bih' % l], p['U%d_bhh' % l],
                    relu_flag=(l == 0))

    readout, a = _readout(h, mol_node_matrix, mol_node_mask, p)
    return readout, a


# speedup vs baseline: 1.0003x; 1.0003x over previous
import functools

import jax
import jax.numpy as jnp
from jax import lax
from jax.experimental import pallas as pl
from jax.experimental.pallas import tpu as pltpu
from jax.experimental.pallas import tpu_sc as plsc

_N, _E, _M = 2048, 8192, 64
_H, _HE = 256, 128
_D = 512
_PW = 272
_ACC_ROWS = 2176
_TRASH = _N
_NC, _NS = 2, 16

_SC_MESH = dict(core_axis_name="c", subcore_axis_name="s",
                num_cores=_NC, num_subcores=_NS)


def _lrelu(x):
    return jnp.where(x >= 0, x, 0.01 * x)


def _gru_math(x, h, wih, whh, bih, bhh):
    gi = jnp.dot(x, wih, preferred_element_type=jnp.float32) + bih
    gh = jnp.dot(h, whh, preferred_element_type=jnp.float32) + bhh
    r = jax.nn.sigmoid(gi[:, :_H] + gh[:, :_H])
    z = jax.nn.sigmoid(gi[:, _H:2 * _H] + gh[:, _H:2 * _H])
    n = jnp.tanh(gi[:, 2 * _H:] + r * gh[:, 2 * _H:])
    return (1.0 - z) * n + z * h


def _proj_body(x_ref, w_ref, b_ref, o_ref):
    o_ref[...] = _lrelu(
        jnp.dot(x_ref[...], w_ref[...], preferred_element_type=jnp.float32)
        + b_ref[...])


def _proj(x, w, b, bm):
    n, k = x.shape
    c = w.shape[1]
    return pl.pallas_call(
        _proj_body,
        grid=(n // bm,),
        in_specs=[pl.BlockSpec((bm, k), lambda i: (i, 0)),
                  pl.BlockSpec((k, c), lambda i: (0, 0)),
                  pl.BlockSpec((1, c), lambda i: (0, 0))],
        out_specs=pl.BlockSpec((bm, c), lambda i: (i, 0)),
        out_shape=jax.ShapeDtypeStruct((n, c), jnp.float32),
    )(x, w, b.reshape(1, c))


def _node_table_body(h_ref, w_ref, o_ref):
    o_ref[...] = jnp.dot(h_ref[...], w_ref[0],
                         preferred_element_type=jnp.float32)


def _node_table(h, w2, bm=256):
    return pl.pallas_call(
        _node_table_body,
        grid=(2, _N // bm),
        in_specs=[pl.BlockSpec((bm, _H), lambda g, i: (i, 0)),
                  pl.BlockSpec((1, _H, _D), lambda g, i: (g, 0, 0))],
        out_specs=pl.BlockSpec((bm, _D), lambda g, i: (g * (_N // bm) + i, 0)),
        out_shape=jax.ShapeDtypeStruct((2 * _N, _D), jnp.float32),
    )(h, w2)


def _edge_fuse_body(g0_ref, g1_ref, e_ref, we_ref, b_ref,
                    pay_ref, eo_ref, msum_ref):
    z = (g0_ref[...] + g1_ref[...]
         + jnp.dot(e_ref[...], we_ref[...], preferred_element_type=jnp.float32)
         + b_ref[...])
    msg = _lrelu(z[:, :_H])
    eo_ref[...] = _lrelu(z[:, _H:_H + _HE])
    w = jnp.exp(z[:, _H + _HE:_H + _HE + 1])
    bm = msg.shape[0]
    pay_ref[...] = jnp.concatenate(
        [w * msg, w, jnp.zeros((bm, _PW - _H - 1), jnp.float32)], axis=1)

    @pl.when(pl.program_id(0) == 0)
    def _():
        msum_ref[...] = jnp.zeros_like(msum_ref)
    msum_ref[...] += jnp.sum(msg, axis=0, keepdims=True)


def _edge_fuse(gat, e, we, b400, bm=512):
    nb = _E // bm
    return pl.pallas_call(
        _edge_fuse_body,
        grid=(nb,),
        in_specs=[pl.BlockSpec((bm, _D), lambda i: (i, 0)),
                  pl.BlockSpec((bm, _D), lambda i: (i + nb, 0)),
                  pl.BlockSpec((bm, _HE), lambda i: (i, 0)),
                  pl.BlockSpec((_HE, _D), lambda i: (0, 0)),
                  pl.BlockSpec((1, _D), lambda i: (0, 0))],
        out_specs=[pl.BlockSpec((bm, _PW), lambda i: (i, 0)),
                   pl.BlockSpec((bm, _HE), lambda i: (i, 0)),
                   pl.BlockSpec((1, _H), lambda i: (0, 0))],
        out_shape=[jax.ShapeDtypeStruct((_E, _PW), jnp.float32),
                   jax.ShapeDtypeStruct((_E, _HE), jnp.float32),
                   jax.ShapeDtypeStruct((1, _H), jnp.float32)],
    )(gat, gat, e, we, b400.reshape(1, _D))


def _nem_matmul_body(nem_ref, p_ref, o_ref):
    @pl.when(pl.program_id(1) == 0)
    def _():
        o_ref[...] = jnp.zeros_like(o_ref)
    o_ref[...] += jnp.dot(nem_ref[...].astype(jnp.bfloat16),
                          p_ref[...].astype(jnp.bfloat16),
                          preferred_element_type=jnp.float32)


def _tc_scatter(nem, payload, bm=256, bk=512):
    return pl.pallas_call(
        _nem_matmul_body,
        grid=(_N // bm, _E // bk),
        in_specs=[pl.BlockSpec((bm, bk), lambda i, k: (i, k)),
                  pl.BlockSpec((bk, _PW), lambda i, k: (k, 0))],
        out_specs=pl.BlockSpec((bm, _PW), lambda i, k: (i, 0)),
        out_shape=jax.ShapeDtypeStruct((_N, _PW), jnp.float32),
    )(nem, payload)


def _update_body(relu_flag, acc_ref, h_ref, msum_ref,
                 wih_ref, whh_ref, bih_ref, bhh_ref, o_ref):
    acc = acc_ref[...]
    numer = acc[:, :_H]
    den = acc[:, _H:_H + 1]
    ctx = jnp.where(den > 0.0, numer / den, msum_ref[...] * (1.0 / _E))
    hn = _gru_math(ctx, h_ref[...], wih_ref[...], whh_ref[...],
                   bih_ref[...], bhh_ref[...])
    if relu_flag:
        hn = jnp.maximum(hn, 0.0)
    o_ref[...] = hn


def _update(acc, h, msum, wih, whh, bih, bhh, relu_flag, bm=256):
    return pl.pallas_call(
        functools.partial(_update_body, relu_flag),
        grid=(_N // bm,),
        in_specs=[pl.BlockSpec((bm, _PW), lambda i: (i, 0)),
                  pl.BlockSpec((bm, _H), lambda i: (i, 0)),
                  pl.BlockSpec((1, _H), lambda i: (0, 0)),
                  pl.BlockSpec((_H, 3 * _H), lambda i: (0, 0)),
                  pl.BlockSpec((_H, 3 * _H), lambda i: (0, 0)),
                  pl.BlockSpec((1, 3 * _H), lambda i: (0, 0)),
                  pl.BlockSpec((1, 3 * _H), lambda i: (0, 0))],
        out_specs=pl.BlockSpec((bm, _H), lambda i: (i, 0)),
        out_shape=jax.ShapeDtypeStruct((_N, _H), jnp.float32),
    )(acc, h, msum, wih, whh, bih.reshape(1, -1), bhh.reshape(1, -1))


def _readout_body(h_ref, mnm_ref, mask_ref, wq_ref, bq_ref, wk_ref, bk_ref,
                  wv_ref, bv_ref, wih_ref, whh_ref, bih_ref, bhh_ref,
                  wo_ref, bo_ref, ro_ref, a_ref):
    h = h_ref[...]
    mnm = mnm_ref[...]
    mask = mask_ref[...]
    deg = jnp.clip(jnp.sum(mnm, axis=1, keepdims=True), 1.0, None)
    m = jnp.dot(mnm, h, preferred_element_type=jnp.float32) / deg
    kx = jnp.dot(h, wk_ref[...], preferred_element_type=jnp.float32) + bk_ref[...]
    vx = jnp.dot(h, wv_ref[...], preferred_element_type=jnp.float32) + bv_ref[...]
    a = None
    for _ in range(2):
        q = jnp.dot(m, wq_ref[...], preferred_element_type=jnp.float32) + bq_ref[...]
        logits = lax.dot_general(q, kx, (((1,), (1,)), ((), ())),
                                 preferred_element_type=jnp.float32) * (1.0 / 16.0)
        logits = logits + mask
        mx = jnp.max(logits, axis=1, keepdims=True)
        ex = jnp.exp(logits - mx)
        a = ex / jnp.sum(ex, axis=1, keepdims=True)
        c = jnp.dot(a, vx, preferred_element_type=jnp.float32)
        m = _gru_math(c, m, wih_ref[...], whh_ref[...], bih_ref[...],
                      bhh_ref[...])
    ro_ref[...] = jnp.maximum(
        jnp.dot(m, wo_ref[...], preferred_element_type=jnp.float32)
        + bo_ref[...], 0.0)
    a_ref[...] = a


def _readout(h, mnm, mask, p):
    args = [h, mnm, mask,
            p['R_q'][0], p['R_q'][1].reshape(1, -1),
            p['R_k'][0], p['R_k'][1].reshape(1, -1),
            p['R_v'][0], p['R_v'][1].reshape(1, -1),
            p['R_Wih'], p['R_Whh'],
            p['R_bih'].reshape(1, -1), p['R_bhh'].reshape(1, -1),
            p['R_out'][0], p['R_out'][1].reshape(1, -1)]
    return pl.pallas_call(
        _readout_body,
        out_shape=[jax.ShapeDtypeStruct((_M, _H), jnp.float32),
                   jax.ShapeDtypeStruct((_M, _N), jnp.float32)],
    )(*args)


def _gather_body(tbl_hbm, idx_hbm, out_hbm, ibuf, rbuf, sem):
    c = lax.axis_index("c")
    s = lax.axis_index("s")
    wid = s * _NC + c
    per = (2 * _E) // (_NC * _NS)
    base0 = wid * per
    for ch in range(per // 128):
        base = base0 + ch * 128
        pltpu.sync_copy(idx_hbm.at[pl.ds(base, 128)], ibuf)
        pltpu.async_copy(tbl_hbm.at[ibuf], rbuf, sem).wait()
        pltpu.sync_copy(rbuf, out_hbm.at[pl.ds(base, 128)])


def _sc_gather(tbl, idx2):
    return pl.kernel(
        _gather_body,
        out_type=jax.ShapeDtypeStruct((2 * _E, _D), jnp.float32),
        mesh=plsc.VectorSubcoreMesh(**_SC_MESH),
        scratch_types=[pltpu.VMEM((128,), jnp.int32),
                       pltpu.VMEM((128, _D), jnp.float32),
                       pltpu.SemaphoreType.DMA],
    )(tbl, idx2)


def _scatter_body(pay_hbm, iu_hbm, iv_hbm, out_hbm, pbuf, ibuf, acc_sh):
    c = lax.axis_index("c")
    s = lax.axis_index("s")
    zr = _ACC_ROWS // _NS
    zero16 = jnp.zeros((16,), jnp.float32)

    def zrow(r, carry):
        for k in range(_PW // 16):
            pbuf[r, pl.ds(k * 16, 16)] = zero16
        return carry

    lax.fori_loop(0, 128, zrow, 0)
    pltpu.sync_copy(pbuf.at[pl.ds(0, 128)], acc_sh.at[pl.ds(s * zr, 128)])
    pltpu.sync_copy(pbuf.at[pl.ds(0, zr - 128)],
                    acc_sh.at[pl.ds(s * zr + 128, zr - 128)])
    plsc.subcore_barrier()

    per = _E // (_NC * _NS)
    base = c * (_E // _NC) + s * per
    for ch in range(per // 128):
        pltpu.sync_copy(pay_hbm.at[pl.ds(base + ch * 128, 128)], pbuf)
        for ih in (iu_hbm, iv_hbm):
            pltpu.sync_copy(ih.at[pl.ds(base + ch * 128, 128)], ibuf)
            pltpu.sync_copy(pbuf, acc_sh.at[ibuf], add=True)
    plsc.subcore_barrier()
    pltpu.sync_copy(acc_sh.at[pl.ds(s * zr, zr)],
                    out_hbm.at[c].at[pl.ds(s * zr, zr)])


def _sc_scatter(payload, idx_u, idx_v):
    return pl.kernel(
        _scatter_body,
        out_type=jax.ShapeDtypeStruct((_NC, _ACC_ROWS, _PW), jnp.float32),
        mesh=plsc.VectorSubcoreMesh(**_SC_MESH),
        scratch_types=[pltpu.VMEM((128, _PW), jnp.float32),
                       pltpu.VMEM((128,), jnp.int32),
                       pltpu.VMEM_SHARED((_ACC_ROWS, _PW), jnp.float32)],
    )(payload, idx_u, idx_v)


def _pack_layer(p, l):
    wm, bm_ = p['M%d_msg' % l]
    wa, ba = p['M%d_att' % l]
    we, be = p['M%d_edge' % l]

    def cols(lo, hi):
        return jnp.concatenate(
            [wm[lo:hi], we[lo:hi], wa[lo:hi],
             jnp.zeros((hi - lo, _D - _H - _HE - 1), jnp.float32)], axis=1)

    w2 = jnp.stack([cols(0, _H), cols(_H, 2 * _H)])
    wee = cols(2 * _H, 2 * _H + _HE)
    b400 = jnp.concatenate(
        [bm_, be, ba, jnp.zeros((_D - _H - _HE - 1,), jnp.float32)])
    return w2, wee, b400


def kernel(node_features, edge_features, us, vs, mol_node_matrix,
           mol_node_mask, node_edge_matrix, node_edge_mask, params):
    p = params
    us32 = us.astype(jnp.int32)
    vs32 = vs.astype(jnp.int32)
    idx2 = jnp.concatenate([us32, vs32 + _N])
    idx_v = jnp.where(us32 == vs32, _TRASH, vs32)

    h = _proj(node_features, p['FC_N'][0], p['FC_N'][1], bm=256)
    e = _proj(edge_features, p['FC_E'][0], p['FC_E'][1], bm=512)

    for l in range(2):
        w2, wee, b400 = _pack_layer(p, l)
        tbl = _node_table(h, w2)
        gat = _sc_gather(tbl, idx2)
        payload, e, msum = _edge_fuse(gat, e, wee, b400)
        acc = _tc_scatter(node_edge_matrix, payload)
        h = _update(acc, h, msum,
                    p['U%d_Wih' % l], p['U%d_Whh' % l],
                    p['U%d_bih' % l], p['U%d_bhh' % l],
                    relu_flag=(l == 0))

    readout, a = _readout(h, mol_node_matrix, mol_node_mask, p)
    return readout, a


# speedup vs baseline: 1.6888x; 1.6883x over previous
import functools

import jax
import jax.numpy as jnp
from jax import lax
from jax.experimental import pallas as pl
from jax.experimental.pallas import tpu as pltpu
from jax.experimental.pallas import tpu_sc as plsc

_N, _E, _M = 2048, 8192, 64
_H, _HE = 256, 128
_D = 512
_PW = 272
_ACC_ROWS = 2176
_TRASH = _N
_NC, _NS = 2, 16

_SC_MESH = dict(core_axis_name="c", subcore_axis_name="s",
                num_cores=_NC, num_subcores=_NS)


def _lrelu(x):
    return jnp.where(x >= 0, x, 0.01 * x)


def _gru_math(x, h, wih, whh, bih, bhh):
    gi = jnp.dot(x, wih, preferred_element_type=jnp.float32) + bih
    gh = jnp.dot(h, whh, preferred_element_type=jnp.float32) + bhh
    r = jax.nn.sigmoid(gi[:, :_H] + gh[:, :_H])
    z = jax.nn.sigmoid(gi[:, _H:2 * _H] + gh[:, _H:2 * _H])
    n = jnp.tanh(gi[:, 2 * _H:] + r * gh[:, 2 * _H:])
    return (1.0 - z) * n + z * h


def _proj_body(x_ref, w_ref, b_ref, o_ref):
    o_ref[...] = _lrelu(
        jnp.dot(x_ref[...], w_ref[...], preferred_element_type=jnp.float32)
        + b_ref[...])


def _proj(x, w, b, bm):
    n, k = x.shape
    c = w.shape[1]
    return pl.pallas_call(
        _proj_body,
        grid=(n // bm,),
        in_specs=[pl.BlockSpec((bm, k), lambda i: (i, 0)),
                  pl.BlockSpec((k, c), lambda i: (0, 0)),
                  pl.BlockSpec((1, c), lambda i: (0, 0))],
        out_specs=pl.BlockSpec((bm, c), lambda i: (i, 0)),
        out_shape=jax.ShapeDtypeStruct((n, c), jnp.float32),
    )(x, w, b.reshape(1, c))


def _node_table_body(h_ref, w_ref, o_ref):
    o_ref[...] = jnp.dot(h_ref[...], w_ref[0],
                         preferred_element_type=jnp.float32)


def _node_table(h, w2, bm=256):
    return pl.pallas_call(
        _node_table_body,
        grid=(2, _N // bm),
        in_specs=[pl.BlockSpec((bm, _H), lambda g, i: (i, 0)),
                  pl.BlockSpec((1, _H, _D), lambda g, i: (g, 0, 0))],
        out_specs=pl.BlockSpec((bm, _D), lambda g, i: (g * (_N // bm) + i, 0)),
        out_shape=jax.ShapeDtypeStruct((2 * _N, _D), jnp.float32),
    )(h, w2)


def _edge_fuse_body(g0_ref, g1_ref, e_ref, we_ref, b_ref,
                    pay_ref, eo_ref, msum_ref):
    z = (g0_ref[...] + g1_ref[...]
         + jnp.dot(e_ref[...], we_ref[...], preferred_element_type=jnp.float32)
         + b_ref[...])
    msg = _lrelu(z[:, :_H])
    eo_ref[...] = _lrelu(z[:, _H:_H + _HE])
    w = jnp.exp(z[:, _H + _HE:_H + _HE + 1])
    bm = msg.shape[0]
    pay_ref[...] = jnp.concatenate(
        [w * msg, w, jnp.zeros((bm, _PW - _H - 1), jnp.float32)],
        axis=1).astype(jnp.bfloat16)

    @pl.when(pl.program_id(0) == 0)
    def _():
        msum_ref[...] = jnp.zeros_like(msum_ref)
    msum_ref[...] += jnp.sum(msg, axis=0, keepdims=True)


def _edge_fuse(gat, e, we, b400, bm=512):
    nb = _E // bm
    return pl.pallas_call(
        _edge_fuse_body,
        grid=(nb,),
        in_specs=[pl.BlockSpec((bm, _D), lambda i: (i, 0)),
                  pl.BlockSpec((bm, _D), lambda i: (i + nb, 0)),
                  pl.BlockSpec((bm, _HE), lambda i: (i, 0)),
                  pl.BlockSpec((_HE, _D), lambda i: (0, 0)),
                  pl.BlockSpec((1, _D), lambda i: (0, 0))],
        out_specs=[pl.BlockSpec((bm, _PW), lambda i: (i, 0)),
                   pl.BlockSpec((bm, _HE), lambda i: (i, 0)),
                   pl.BlockSpec((1, _H), lambda i: (0, 0))],
        out_shape=[jax.ShapeDtypeStruct((_E, _PW), jnp.bfloat16),
                   jax.ShapeDtypeStruct((_E, _HE), jnp.float32),
                   jax.ShapeDtypeStruct((1, _H), jnp.float32)],
    )(gat, gat, e, we, b400.reshape(1, _D))


def _nem_matmul_body(nem_ref, p_ref, o_ref):
    o_ref[...] = jnp.dot(nem_ref[...].astype(jnp.bfloat16), p_ref[...],
                         preferred_element_type=jnp.float32)


def _tc_scatter(nem, payload, bm=256):
    return pl.pallas_call(
        _nem_matmul_body,
        grid=(_N // bm,),
        in_specs=[pl.BlockSpec((bm, _E), lambda i: (i, 0)),
                  pl.BlockSpec((_E, _PW), lambda i: (0, 0))],
        out_specs=pl.BlockSpec((bm, _PW), lambda i: (i, 0)),
        out_shape=jax.ShapeDtypeStruct((_N, _PW), jnp.float32),
    )(nem, payload)


def _update_body(relu_flag, acc_ref, h_ref, msum_ref,
                 wih_ref, whh_ref, bih_ref, bhh_ref, o_ref):
    acc = acc_ref[...]
    numer = acc[:, :_H]
    den = acc[:, _H:_H + 1]
    ctx = jnp.where(den > 0.0, numer / den, msum_ref[...] * (1.0 / _E))
    hn = _gru_math(ctx, h_ref[...], wih_ref[...], whh_ref[...],
                   bih_ref[...], bhh_ref[...])
    if relu_flag:
        hn = jnp.maximum(hn, 0.0)
    o_ref[...] = hn


def _update(acc, h, msum, wih, whh, bih, bhh, relu_flag, bm=256):
    return pl.pallas_call(
        functools.partial(_update_body, relu_flag),
        grid=(_N // bm,),
        in_specs=[pl.BlockSpec((bm, _PW), lambda i: (i, 0)),
                  pl.BlockSpec((bm, _H), lambda i: (i, 0)),
                  pl.BlockSpec((1, _H), lambda i: (0, 0)),
                  pl.BlockSpec((_H, 3 * _H), lambda i: (0, 0)),
                  pl.BlockSpec((_H, 3 * _H), lambda i: (0, 0)),
                  pl.BlockSpec((1, 3 * _H), lambda i: (0, 0)),
                  pl.BlockSpec((1, 3 * _H), lambda i: (0, 0))],
        out_specs=pl.BlockSpec((bm, _H), lambda i: (i, 0)),
        out_shape=jax.ShapeDtypeStruct((_N, _H), jnp.float32),
    )(acc, h, msum, wih, whh, bih.reshape(1, -1), bhh.reshape(1, -1))


def _readout_body(h_ref, mnm_ref, mask_ref, wq_ref, bq_ref, wk_ref, bk_ref,
                  wv_ref, bv_ref, wih_ref, whh_ref, bih_ref, bhh_ref,
                  wo_ref, bo_ref, ro_ref, a_ref):
    h = h_ref[...]
    mnm = mnm_ref[...]
    mask = mask_ref[...]
    deg = jnp.clip(jnp.sum(mnm, axis=1, keepdims=True), 1.0, None)
    m = jnp.dot(mnm, h, preferred_element_type=jnp.float32) / deg
    kx = jnp.dot(h, wk_ref[...], preferred_element_type=jnp.float32) + bk_ref[...]
    vx = jnp.dot(h, wv_ref[...], preferred_element_type=jnp.float32) + bv_ref[...]
    a = None
    for _ in range(2):
        q = jnp.dot(m, wq_ref[...], preferred_element_type=jnp.float32) + bq_ref[...]
        logits = lax.dot_general(q, kx, (((1,), (1,)), ((), ())),
                                 preferred_element_type=jnp.float32) * (1.0 / 16.0)
        logits = logits + mask
        mx = jnp.max(logits, axis=1, keepdims=True)
        ex = jnp.exp(logits - mx)
        a = ex / jnp.sum(ex, axis=1, keepdims=True)
        c = jnp.dot(a, vx, preferred_element_type=jnp.float32)
        m = _gru_math(c, m, wih_ref[...], whh_ref[...], bih_ref[...],
                      bhh_ref[...])
    ro_ref[...] = jnp.maximum(
        jnp.dot(m, wo_ref[...], preferred_element_type=jnp.float32)
        + bo_ref[...], 0.0)
    a_ref[...] = a


def _readout(h, mnm, mask, p):
    args = [h, mnm, mask,
            p['R_q'][0], p['R_q'][1].reshape(1, -1),
            p['R_k'][0], p['R_k'][1].reshape(1, -1),
            p['R_v'][0], p['R_v'][1].reshape(1, -1),
            p['R_Wih'], p['R_Whh'],
            p['R_bih'].reshape(1, -1), p['R_bhh'].reshape(1, -1),
            p['R_out'][0], p['R_out'][1].reshape(1, -1)]
    return pl.pallas_call(
        _readout_body,
        out_shape=[jax.ShapeDtypeStruct((_M, _H), jnp.float32),
                   jax.ShapeDtypeStruct((_M, _N), jnp.float32)],
    )(*args)


def _gather_body(tbl_hbm, idx_hbm, out_hbm, ibuf, rbuf, sem):
    c = lax.axis_index("c")
    s = lax.axis_index("s")
    wid = s * _NC + c
    per = (2 * _E) // (_NC * _NS)
    base0 = wid * per
    for ch in range(per // 128):
        base = base0 + ch * 128
        pltpu.sync_copy(idx_hbm.at[pl.ds(base, 128)], ibuf)
        pltpu.async_copy(tbl_hbm.at[ibuf], rbuf, sem).wait()
        pltpu.sync_copy(rbuf, out_hbm.at[pl.ds(base, 128)])


def _sc_gather(tbl, idx2):
    return pl.kernel(
        _gather_body,
        out_type=jax.ShapeDtypeStruct((2 * _E, _D), jnp.float32),
        mesh=plsc.VectorSubcoreMesh(**_SC_MESH),
        scratch_types=[pltpu.VMEM((128,), jnp.int32),
                       pltpu.VMEM((128, _D), jnp.float32),
                       pltpu.SemaphoreType.DMA],
    )(tbl, idx2)


def _scatter_body(pay_hbm, iu_hbm, iv_hbm, out_hbm, pbuf, ibuf, acc_sh):
    c = lax.axis_index("c")
    s = lax.axis_index("s")
    zr = _ACC_ROWS // _NS
    zero16 = jnp.zeros((16,), jnp.float32)

    def zrow(r, carry):
        for k in range(_PW // 16):
            pbuf[r, pl.ds(k * 16, 16)] = zero16
        return carry

    lax.fori_loop(0, 128, zrow, 0)
    pltpu.sync_copy(pbuf.at[pl.ds(0, 128)], acc_sh.at[pl.ds(s * zr, 128)])
    pltpu.sync_copy(pbuf.at[pl.ds(0, zr - 128)],
                    acc_sh.at[pl.ds(s * zr + 128, zr - 128)])
    plsc.subcore_barrier()

    per = _E // (_NC * _NS)
    base = c * (_E // _NC) + s * per
    for ch in range(per // 128):
        pltpu.sync_copy(pay_hbm.at[pl.ds(base + ch * 128, 128)], pbuf)
        for ih in (iu_hbm, iv_hbm):
            pltpu.sync_copy(ih.at[pl.ds(base + ch * 128, 128)], ibuf)
            pltpu.sync_copy(pbuf, acc_sh.at[ibuf], add=True)
    plsc.subcore_barrier()
    pltpu.sync_copy(acc_sh.at[pl.ds(s * zr, zr)],
                    out_hbm.at[c].at[pl.ds(s * zr, zr)])


def _sc_scatter(payload, idx_u, idx_v):
    return pl.kernel(
        _scatter_body,
        out_type=jax.ShapeDtypeStruct((_NC, _ACC_ROWS, _PW), jnp.float32),
        mesh=plsc.VectorSubcoreMesh(**_SC_MESH),
        scratch_types=[pltpu.VMEM((128, _PW), jnp.float32),
                       pltpu.VMEM((128,), jnp.int32),
                       pltpu.VMEM_SHARED((_ACC_ROWS, _PW), jnp.float32)],
    )(payload, idx_u, idx_v)


def _pack_layer(p, l):
    wm, bm_ = p['M%d_msg' % l]
    wa, ba = p['M%d_att' % l]
    we, be = p['M%d_edge' % l]

    def cols(lo, hi):
        return jnp.concatenate(
            [wm[lo:hi], we[lo:hi], wa[lo:hi],
             jnp.zeros((hi - lo, _D - _H - _HE - 1), jnp.float32)], axis=1)

    w2 = jnp.stack([cols(0, _H), cols(_H, 2 * _H)])
    wee = cols(2 * _H, 2 * _H + _HE)
    b400 = jnp.concatenate(
        [bm_, be, ba, jnp.zeros((_D - _H - _HE - 1,), jnp.float32)])
    return w2, wee, b400


def kernel(node_features, edge_features, us, vs, mol_node_matrix,
           mol_node_mask, node_edge_matrix, node_edge_mask, params):
    p = params
    us32 = us.astype(jnp.int32)
    vs32 = vs.astype(jnp.int32)
    idx2 = jnp.concatenate([us32, vs32 + _N])
    idx_v = jnp.where(us32 == vs32, _TRASH, vs32)

    h = _proj(node_features, p['FC_N'][0], p['FC_N'][1], bm=256)
    e = _proj(edge_features, p['FC_E'][0], p['FC_E'][1], bm=512)

    for l in range(2):
        w2, wee, b400 = _pack_layer(p, l)
        tbl = _node_table(h, w2)
        gat = _sc_gather(tbl, idx2)
        payload, e, msum = _edge_fuse(gat, e, wee, b400)
        acc = _tc_scatter(node_edge_matrix, payload)
        h = _update(acc, h, msum,
                    p['U%d_Wih' % l], p['U%d_Whh' % l],
                    p['U%d_bih' % l], p['U%d_bhh' % l],
                    relu_flag=(l == 0))

    readout, a = _readout(h, mol_node_matrix, mol_node_mask, p)
    return readout, a
